# Initial kernel scaffold; baseline (speedup 1.0000x reference)
import jax
import jax.numpy as jnp
from jax.experimental import pallas as pl


def kernel(x, edge_index, W1l, b1l, W1r, bn_gamma, bn_beta, bn_mean, bn_var, W2l, b2l, W2r):
    raise NotImplementedError("write your pallas kernel here")


# speedup vs baseline: 3.2020x; 3.2020x over previous
import functools

import jax
import jax.numpy as jnp
from jax import lax
from jax.experimental import pallas as pl
from jax.experimental.pallas import tpu as pltpu
from jax.experimental.pallas import tpu_sc as plsc

N = 10000
E = 320000
IN = 128
H = 256
OUT = 128

NC = 2
NS = 16
NW = NC * NS

NP = 10240
RPT = NP // NS
EP = 327680
EW = EP // NW
K = 128
ZR = 128

BR = 512
NB = NP // BR


def _make_sc_agg(F, with_deg):
    mesh = plsc.VectorSubcoreMesh(
        core_axis_name="c", subcore_axis_name="s", num_cores=NC, num_subcores=NS
    )
    out_type = [jax.ShapeDtypeStruct((NC, NP, F), jnp.float32)]
    scratch = [
        pltpu.VMEM((K,), jnp.int32),
        pltpu.VMEM((K,), jnp.int32),
        pltpu.VMEM((K, F), jnp.float32),
        pltpu.VMEM((ZR, F), jnp.float32),
        pltpu.VMEM_SHARED((NP, F), jnp.float32),
        pltpu.SemaphoreType.DMA,
    ]
    if with_deg:
        out_type.append(jax.ShapeDtypeStruct((NC, NP), jnp.float32))
        scratch += [
            pltpu.VMEM((K,), jnp.float32),
            pltpu.VMEM((RPT,), jnp.float32),
            pltpu.VMEM_SHARED((NP,), jnp.float32),
        ]

    def body(table, srcp, dstp, *refs):
        if with_deg:
            out, dout, sidx, didx, rows, zbuf, acc, sem, ones, zdeg, dacc = refs
        else:
            out, sidx, didx, rows, zbuf, acc, sem = refs
        c = lax.axis_index("c")
        s = lax.axis_index("s")
        wid = s * NC + c

        def zfill(k, _):
            zbuf[k // (F // 16), pl.ds((k % (F // 16)) * 16, 16)] = jnp.zeros(
                (16,), jnp.float32
            )
            return 0

        lax.fori_loop(0, ZR * (F // 16), zfill, 0)
        for k in range(RPT // ZR):
            pltpu.sync_copy(zbuf, acc.at[pl.ds(s * RPT + k * ZR, ZR)])
        if with_deg:
            def ofill(k, _):
                ones[pl.ds(k * 16, 16)] = jnp.ones((16,), jnp.float32)
                return 0

            lax.fori_loop(0, K // 16, ofill, 0)

            def dzfill(k, _):
                zdeg[pl.ds(k * 16, 16)] = jnp.zeros((16,), jnp.float32)
                return 0

            lax.fori_loop(0, RPT // 16, dzfill, 0)
            pltpu.sync_copy(zdeg, dacc.at[pl.ds(s * RPT, RPT)])
        plsc.subcore_barrier()

        ebase = wid * EW

        def chunk(t, _):
            b = ebase + t * K
            pltpu.sync_copy(srcp.at[pl.ds(b, K)], sidx)
            pltpu.sync_copy(dstp.at[pl.ds(b, K)], didx)
            pltpu.async_copy(table.at[sidx], rows, sem).wait()
            pltpu.sync_copy(rows, acc.at[didx], add=True)
            if with_deg:
                pltpu.sync_copy(ones, dacc.at[didx], add=True)
            return 0

        lax.fori_loop(0, EW // K, chunk, 0)
        plsc.subcore_barrier()

        pltpu.sync_copy(acc.at[pl.ds(s * RPT, RPT)], out.at[c, pl.ds(s * RPT, RPT)])
        if with_deg:
            pltpu.sync_copy(
                dacc.at[pl.ds(s * RPT, RPT)], dout.at[c, pl.ds(s * RPT, RPT)]
            )

    return pl.kernel(
        body, out_type=tuple(out_type), mesh=mesh, scratch_types=tuple(scratch)
    )


def _dot(a, b):
    return jnp.dot(a, b, preferred_element_type=jnp.float32,
                   precision=lax.Precision.HIGHEST)


def _tc_stage1(part1, deg3, x, w1lT, w1rT, b1l, scale, shift, w2lT, w2rT, b2l):
    def body(p_ref, d_ref, x_ref, w1l_ref, w1r_ref, b1_ref, sc_ref, sh_ref,
             w2l_ref, w2r_ref, b2_ref, pout_ref, rout_ref, inv_ref):
        deg = d_ref[0, 0] + d_ref[0, 1]
        inv = 1.0 / jnp.maximum(deg, 1.0)
        inv_ref[...] = inv.reshape(1, 1, BR)
        agg = (p_ref[0] + p_ref[1]) * inv[:, None]
        t = _dot(agg, w1l_ref[...]) + _dot(x_ref[...], w1r_ref[...]) + b1_ref[...]
        h = jnp.maximum(t * sc_ref[...] + sh_ref[...], 0.0)
        pout_ref[...] = _dot(h, w2l_ref[...])
        rout_ref[...] = _dot(h, w2r_ref[...]) + b2_ref[...]

    full = lambda i: (0, 0)
    return pl.pallas_call(
        body,
        grid=(NB,),
        in_specs=[
            pl.BlockSpec((NC, BR, IN), lambda i: (0, i, 0)),
            pl.BlockSpec((1, NC, BR), lambda i: (i, 0, 0)),
            pl.BlockSpec((BR, IN), lambda i: (i, 0)),
            pl.BlockSpec((IN, H), full),
            pl.BlockSpec((IN, H), full),
            pl.BlockSpec((1, H), full),
            pl.BlockSpec((1, H), full),
            pl.BlockSpec((1, H), full),
            pl.BlockSpec((H, OUT), full),
            pl.BlockSpec((H, OUT), full),
            pl.BlockSpec((1, OUT), full),
        ],
        out_specs=[
            pl.BlockSpec((BR, OUT), lambda i: (i, 0)),
            pl.BlockSpec((BR, OUT), lambda i: (i, 0)),
            pl.BlockSpec((1, 1, BR), lambda i: (i, 0, 0)),
        ],
        out_shape=[
            jax.ShapeDtypeStruct((NP, OUT), jnp.float32),
            jax.ShapeDtypeStruct((NP, OUT), jnp.float32),
            jax.ShapeDtypeStruct((NB, 1, BR), jnp.float32),
        ],
    )(part1, deg3, x, w1lT, w1rT, b1l, scale, shift, w2lT, w2rT, b2l)


def _tc_stage2(part2, inv3, r):
    def body(p_ref, i_ref, r_ref, o_ref):
        inv = i_ref[0, 0]
        o_ref[...] = (p_ref[0] + p_ref[1]) * inv[:, None] + r_ref[...]

    return pl.pallas_call(
        body,
        grid=(NB,),
        in_specs=[
            pl.BlockSpec((NC, BR, OUT), lambda i: (0, i, 0)),
            pl.BlockSpec((1, 1, BR), lambda i: (i, 0, 0)),
            pl.BlockSpec((BR, OUT), lambda i: (i, 0)),
        ],
        out_specs=pl.BlockSpec((BR, OUT), lambda i: (i, 0)),
        out_shape=jax.ShapeDtypeStruct((NP, OUT), jnp.float32),
    )(part2, inv3, r)


_sc_agg_deg = _make_sc_agg(IN, True)
_sc_agg = _make_sc_agg(OUT, False)


@jax.jit
def kernel(x, edge_index, W1l, b1l, W1r, bn_gamma, bn_beta, bn_mean, bn_var,
           W2l, b2l, W2r):
    src = edge_index[0]
    dst = edge_index[1]
    src_p = jnp.concatenate([src, jnp.zeros((EP - E,), jnp.int32)])
    dst_p = jnp.concatenate([dst, jnp.full((EP - E,), NP - 1, jnp.int32)])
    x_p = jnp.pad(x, ((0, NP - N), (0, 0)))

    rstd = 1.0 / jnp.sqrt(bn_var + 1e-5)
    scale = (bn_gamma * rstd).reshape(1, H)
    shift = (bn_beta - bn_mean * bn_gamma * rstd).reshape(1, H)

    part1, degp = _sc_agg_deg(x_p, src_p, dst_p)
    deg3 = degp.reshape(NC, NB, BR).transpose(1, 0, 2)

    p, r, inv3 = _tc_stage1(
        part1, deg3, x_p, W1l.T, W1r.T, b1l.reshape(1, H), scale, shift,
        W2l.T, W2r.T, b2l.reshape(1, OUT),
    )

    (part2,) = _sc_agg(p, src_p, dst_p)
    out_p = _tc_stage2(part2, inv3, r)
    return out_p[:N]


# speedup vs baseline: 3.4073x; 1.0641x over previous
import functools

import jax
import jax.numpy as jnp
from jax import lax
from jax.experimental import pallas as pl
from jax.experimental.pallas import tpu as pltpu
from jax.experimental.pallas import tpu_sc as plsc

N = 10000
E = 320000
IN = 128
H = 256
OUT = 128

NC = 2
NS = 16
NW = NC * NS

NP = 10240
RPT = NP // NS
EP = 327680
EW = EP // NW
K = 128
CPW = EW // K
NBUF = 2
ZR = 128

BR = 512
NB = NP // BR


def _make_sc_agg(F, with_deg):
    mesh = plsc.VectorSubcoreMesh(
        core_axis_name="c", subcore_axis_name="s", num_cores=NC, num_subcores=NS
    )
    out_type = [jax.ShapeDtypeStruct((NC, NP, F), jnp.float32)]
    scratch = [
        pltpu.VMEM((2, 1, K), jnp.int32),
        pltpu.VMEM((2, 1, K), jnp.int32),
        pltpu.VMEM((2, K, F), jnp.float32),
        pltpu.VMEM_SHARED((NP, F), jnp.float32),
        pltpu.SemaphoreType.DMA,
    ]
    if with_deg:
        out_type.append(jax.ShapeDtypeStruct((NC, NP), jnp.float32))
        scratch += [
            pltpu.VMEM((K,), jnp.float32),
            pltpu.VMEM((RPT,), jnp.float32),
            pltpu.VMEM_SHARED((NP,), jnp.float32),
        ]

    def body(table, srcp3, dstp3, *refs):
        if with_deg:
            out, dout, sidx, didx, rows, acc, gsem, ones, zdeg, dacc = refs
        else:
            out, sidx, didx, rows, acc, gsem = refs
        c = lax.axis_index("c")
        s = lax.axis_index("s")
        wid = s * NC + c

        def zfill(k, _):
            rows[0, k // (F // 16), pl.ds((k % (F // 16)) * 16, 16)] = jnp.zeros(
                (16,), jnp.float32
            )
            return 0

        lax.fori_loop(0, K * (F // 16), zfill, 0)
        for k in range(RPT // K):
            pltpu.sync_copy(rows.at[0], acc.at[pl.ds(s * RPT + k * K, K)])
        if with_deg:
            def ofill(k, _):
                ones[pl.ds(k * 16, 16)] = jnp.ones((16,), jnp.float32)
                return 0

            lax.fori_loop(0, K // 16, ofill, 0)

            def dzfill(k, _):
                zdeg[pl.ds(k * 16, 16)] = jnp.zeros((16,), jnp.float32)
                return 0

            lax.fori_loop(0, RPT // 16, dzfill, 0)
            pltpu.sync_copy(zdeg, dacc.at[pl.ds(s * RPT, RPT)])
        plsc.subcore_barrier()

        cb = wid * CPW

        pltpu.sync_copy(srcp3.at[cb], sidx.at[0])
        pltpu.sync_copy(dstp3.at[cb], didx.at[0])
        pltpu.async_copy(table.at[sidx.at[0, 0]], rows.at[0], gsem)

        def chunk(t, _):
            b = lax.rem(t, 2)
            bn = lax.rem(t + 1, 2)
            tn = jnp.minimum(t + 1, CPW - 1)
            pltpu.make_async_copy(table.at[sidx.at[b, 0]], rows.at[b], gsem).wait()
            pltpu.sync_copy(srcp3.at[cb + tn], sidx.at[bn])
            pltpu.sync_copy(dstp3.at[cb + tn], didx.at[bn])
            pltpu.async_copy(table.at[sidx.at[bn, 0]], rows.at[bn], gsem)
            pltpu.sync_copy(rows.at[b], acc.at[didx.at[b, 0]], add=True)
            if with_deg:
                pltpu.sync_copy(ones, dacc.at[didx.at[b, 0]], add=True)
            return 0

        lax.fori_loop(0, CPW, chunk, 0)
        pltpu.make_async_copy(table.at[sidx.at[0, 0]], rows.at[0], gsem).wait()
        plsc.subcore_barrier()

        pltpu.sync_copy(acc.at[pl.ds(s * RPT, RPT)], out.at[c, pl.ds(s * RPT, RPT)])
        if with_deg:
            pltpu.sync_copy(
                dacc.at[pl.ds(s * RPT, RPT)], dout.at[c, pl.ds(s * RPT, RPT)]
            )

    return pl.kernel(
        body, out_type=tuple(out_type), mesh=mesh, scratch_types=tuple(scratch)
    )


def _dot(a, b):
    return jnp.dot(a, b, preferred_element_type=jnp.float32,
                   precision=lax.Precision.HIGHEST)


def _tc_stage1(part1, deg3, x, w1lT, w1rT, b1l, scale, shift, w2lT, w2rT, b2l):
    def body(p_ref, d_ref, x_ref, w1l_ref, w1r_ref, b1_ref, sc_ref, sh_ref,
             w2l_ref, w2r_ref, b2_ref, pout_ref, rout_ref, inv_ref):
        deg = d_ref[0, 0] + d_ref[0, 1]
        inv = 1.0 / jnp.maximum(deg, 1.0)
        inv_ref[...] = inv.reshape(1, 1, BR)
        agg = (p_ref[0] + p_ref[1]) * inv[:, None]
        t = _dot(agg, w1l_ref[...]) + _dot(x_ref[...], w1r_ref[...]) + b1_ref[...]
        h = jnp.maximum(t * sc_ref[...] + sh_ref[...], 0.0)
        pout_ref[...] = _dot(h, w2l_ref[...])
        rout_ref[...] = _dot(h, w2r_ref[...]) + b2_ref[...]

    full = lambda i: (0, 0)
    return pl.pallas_call(
        body,
        grid=(NB,),
        in_specs=[
            pl.BlockSpec((NC, BR, IN), lambda i: (0, i, 0)),
            pl.BlockSpec((1, NC, BR), lambda i: (i, 0, 0)),
            pl.BlockSpec((BR, IN), lambda i: (i, 0)),
            pl.BlockSpec((IN, H), full),
            pl.BlockSpec((IN, H), full),
            pl.BlockSpec((1, H), full),
            pl.BlockSpec((1, H), full),
            pl.BlockSpec((1, H), full),
            pl.BlockSpec((H, OUT), full),
            pl.BlockSpec((H, OUT), full),
            pl.BlockSpec((1, OUT), full),
        ],
        out_specs=[
            pl.BlockSpec((BR, OUT), lambda i: (i, 0)),
            pl.BlockSpec((BR, OUT), lambda i: (i, 0)),
            pl.BlockSpec((1, 1, BR), lambda i: (i, 0, 0)),
        ],
        out_shape=[
            jax.ShapeDtypeStruct((NP, OUT), jnp.float32),
            jax.ShapeDtypeStruct((NP, OUT), jnp.float32),
            jax.ShapeDtypeStruct((NB, 1, BR), jnp.float32),
        ],
    )(part1, deg3, x, w1lT, w1rT, b1l, scale, shift, w2lT, w2rT, b2l)


def _tc_stage2(part2, inv3, r):
    def body(p_ref, i_ref, r_ref, o_ref):
        inv = i_ref[0, 0]
        o_ref[...] = (p_ref[0] + p_ref[1]) * inv[:, None] + r_ref[...]

    return pl.pallas_call(
        body,
        grid=(NB,),
        in_specs=[
            pl.BlockSpec((NC, BR, OUT), lambda i: (0, i, 0)),
            pl.BlockSpec((1, 1, BR), lambda i: (i, 0, 0)),
            pl.BlockSpec((BR, OUT), lambda i: (i, 0)),
        ],
        out_specs=pl.BlockSpec((BR, OUT), lambda i: (i, 0)),
        out_shape=jax.ShapeDtypeStruct((NP, OUT), jnp.float32),
    )(part2, inv3, r)


_sc_agg_deg = _make_sc_agg(IN, True)
_sc_agg = _make_sc_agg(OUT, False)


@jax.jit
def kernel(x, edge_index, W1l, b1l, W1r, bn_gamma, bn_beta, bn_mean, bn_var,
           W2l, b2l, W2r):
    src = edge_index[0]
    dst = edge_index[1]
    src_p = jnp.concatenate([src, jnp.zeros((EP - E,), jnp.int32)]).reshape(
        EP // K, 1, K)
    dst_p = jnp.concatenate([dst, jnp.full((EP - E,), NP - 1, jnp.int32)]).reshape(
        EP // K, 1, K)
    x_p = jnp.pad(x, ((0, NP - N), (0, 0)))

    rstd = 1.0 / jnp.sqrt(bn_var + 1e-5)
    scale = (bn_gamma * rstd).reshape(1, H)
    shift = (bn_beta - bn_mean * bn_gamma * rstd).reshape(1, H)

    part1, degp = _sc_agg_deg(x_p, src_p, dst_p)
    deg3 = degp.reshape(NC, NB, BR).transpose(1, 0, 2)

    p, r, inv3 = _tc_stage1(
        part1, deg3, x_p, W1l.T, W1r.T, b1l.reshape(1, H), scale, shift,
        W2l.T, W2r.T, b2l.reshape(1, OUT),
    )

    (part2,) = _sc_agg(p, src_p, dst_p)
    out_p = _tc_stage2(part2, inv3, r)
    return out_p[:N]


# speedup vs baseline: 3.9737x; 1.1662x over previous
import functools

import jax
import jax.numpy as jnp
from jax import lax
from jax.experimental import pallas as pl
from jax.experimental.pallas import tpu as pltpu
from jax.experimental.pallas import tpu_sc as plsc

N = 10000
E = 320000
IN = 128
H = 256
OUT = 128

NC = 2
NS = 16
NW = NC * NS

NP = 10240
RPT = NP // NS
EP = 327680
EW = EP // NW
K = 128
CPW = EW // K
NBUF = 2
ZR = 128

BR = 512
NB = NP // BR


def _make_sc_agg(F, with_deg):
    mesh = plsc.VectorSubcoreMesh(
        core_axis_name="c", subcore_axis_name="s", num_cores=NC, num_subcores=NS
    )
    out_type = [jax.ShapeDtypeStruct((NC, NP, F), jnp.float32)]
    scratch = [
        pltpu.VMEM((3, 2, K), jnp.int32),
        pltpu.VMEM((2, K, F), jnp.float32),
        pltpu.VMEM_SHARED((NP, F), jnp.float32),
        pltpu.SemaphoreType.DMA,
        pltpu.SemaphoreType.DMA,
    ]
    if with_deg:
        out_type.append(jax.ShapeDtypeStruct((NC, NP), jnp.float32))
        scratch += [
            pltpu.SemaphoreType.DMA,
            pltpu.VMEM((K,), jnp.float32),
            pltpu.VMEM((RPT,), jnp.float32),
            pltpu.VMEM_SHARED((NP,), jnp.float32),
        ]

    def body(table, epk, *refs):
        if with_deg:
            out, dout, cidx, rows, acc, gsem, isem, dsem, ones, zdeg, dacc = refs
        else:
            out, cidx, rows, acc, gsem, isem = refs
        c = lax.axis_index("c")
        s = lax.axis_index("s")
        wid = s * NC + c

        def zfill(k, _):
            rows[0, k // (F // 16), pl.ds((k % (F // 16)) * 16, 16)] = jnp.zeros(
                (16,), jnp.float32
            )
            return 0

        lax.fori_loop(0, K * (F // 16), zfill, 0)
        for k in range(RPT // K):
            pltpu.sync_copy(rows.at[0], acc.at[pl.ds(s * RPT + k * K, K)])
        if with_deg:
            def ofill(k, _):
                ones[pl.ds(k * 16, 16)] = jnp.ones((16,), jnp.float32)
                return 0

            lax.fori_loop(0, K // 16, ofill, 0)

            def dzfill(k, _):
                zdeg[pl.ds(k * 16, 16)] = jnp.zeros((16,), jnp.float32)
                return 0

            lax.fori_loop(0, RPT // 16, dzfill, 0)
            pltpu.sync_copy(zdeg, dacc.at[pl.ds(s * RPT, RPT)])
        plsc.subcore_barrier()

        cb = wid * CPW

        pltpu.sync_copy(epk.at[cb], cidx.at[0])
        pltpu.async_copy(epk.at[cb + 1], cidx.at[1], isem)
        pltpu.async_copy(table.at[cidx.at[0, 0]], rows.at[0], gsem)

        def chunk(t, _):
            b = lax.rem(t, 2)
            bn = lax.rem(t + 1, 2)
            s0 = lax.rem(t, 3)
            s1 = lax.rem(t + 1, 3)
            s2 = lax.rem(t + 2, 3)
            tn1 = jnp.minimum(t + 1, CPW - 1)
            tn2 = jnp.minimum(t + 2, CPW - 1)
            pltpu.make_async_copy(table.at[cidx.at[s0, 0]], rows.at[b], gsem).wait()
            pltpu.make_async_copy(epk.at[cb + tn1], cidx.at[s1], isem).wait()
            pltpu.async_copy(table.at[cidx.at[s1, 0]], rows.at[bn], gsem)
            pltpu.async_copy(epk.at[cb + tn2], cidx.at[s2], isem)
            if with_deg:
                dsc = pltpu.async_copy(ones, dacc.at[cidx.at[s0, 1]], dsem, add=True)
            pltpu.sync_copy(rows.at[b], acc.at[cidx.at[s0, 1]], add=True)
            if with_deg:
                dsc.wait()
            return 0

        lax.fori_loop(0, CPW, chunk, 0)
        pltpu.make_async_copy(
            table.at[cidx.at[CPW % 3, 0]], rows.at[CPW % 2], gsem
        ).wait()
        pltpu.make_async_copy(
            epk.at[cb + CPW - 1], cidx.at[(CPW + 1) % 3], isem
        ).wait()
        plsc.subcore_barrier()

        pltpu.sync_copy(acc.at[pl.ds(s * RPT, RPT)], out.at[c, pl.ds(s * RPT, RPT)])
        if with_deg:
            pltpu.sync_copy(
                dacc.at[pl.ds(s * RPT, RPT)], dout.at[c, pl.ds(s * RPT, RPT)]
            )

    return pl.kernel(
        body, out_type=tuple(out_type), mesh=mesh, scratch_types=tuple(scratch)
    )


def _dot(a, b):
    return jnp.dot(a, b, preferred_element_type=jnp.float32,
                   precision=lax.Precision.HIGHEST)


def _tc_stage1(part1, deg3, x, w1lT, w1rT, b1l, scale, shift, w2lT, w2rT, b2l):
    def body(p_ref, d_ref, x_ref, w1l_ref, w1r_ref, b1_ref, sc_ref, sh_ref,
             w2l_ref, w2r_ref, b2_ref, pout_ref, rout_ref, inv_ref):
        deg = d_ref[0, 0] + d_ref[0, 1]
        inv = 1.0 / jnp.maximum(deg, 1.0)
        inv_ref[...] = inv.reshape(1, 1, BR)
        agg = (p_ref[0] + p_ref[1]) * inv[:, None]
        t = _dot(agg, w1l_ref[...]) + _dot(x_ref[...], w1r_ref[...]) + b1_ref[...]
        h = jnp.maximum(t * sc_ref[...] + sh_ref[...], 0.0)
        pout_ref[...] = _dot(h, w2l_ref[...])
        rout_ref[...] = _dot(h, w2r_ref[...]) + b2_ref[...]

    full = lambda i: (0, 0)
    return pl.pallas_call(
        body,
        grid=(NB,),
        in_specs=[
            pl.BlockSpec((NC, BR, IN), lambda i: (0, i, 0)),
            pl.BlockSpec((1, NC, BR), lambda i: (i, 0, 0)),
            pl.BlockSpec((BR, IN), lambda i: (i, 0)),
            pl.BlockSpec((IN, H), full),
            pl.BlockSpec((IN, H), full),
            pl.BlockSpec((1, H), full),
            pl.BlockSpec((1, H), full),
            pl.BlockSpec((1, H), full),
            pl.BlockSpec((H, OUT), full),
            pl.BlockSpec((H, OUT), full),
            pl.BlockSpec((1, OUT), full),
        ],
        out_specs=[
            pl.BlockSpec((BR, OUT), lambda i: (i, 0)),
            pl.BlockSpec((BR, OUT), lambda i: (i, 0)),
            pl.BlockSpec((1, 1, BR), lambda i: (i, 0, 0)),
        ],
        out_shape=[
            jax.ShapeDtypeStruct((NP, OUT), jnp.float32),
            jax.ShapeDtypeStruct((NP, OUT), jnp.float32),
            jax.ShapeDtypeStruct((NB, 1, BR), jnp.float32),
        ],
    )(part1, deg3, x, w1lT, w1rT, b1l, scale, shift, w2lT, w2rT, b2l)


def _tc_stage2(part2, inv3, r):
    def body(p_ref, i_ref, r_ref, o_ref):
        inv = i_ref[0, 0]
        o_ref[...] = (p_ref[0] + p_ref[1]) * inv[:, None] + r_ref[...]

    return pl.pallas_call(
        body,
        grid=(NB,),
        in_specs=[
            pl.BlockSpec((NC, BR, OUT), lambda i: (0, i, 0)),
            pl.BlockSpec((1, 1, BR), lambda i: (i, 0, 0)),
            pl.BlockSpec((BR, OUT), lambda i: (i, 0)),
        ],
        out_specs=pl.BlockSpec((BR, OUT), lambda i: (i, 0)),
        out_shape=jax.ShapeDtypeStruct((NP, OUT), jnp.float32),
    )(part2, inv3, r)


_sc_agg_deg = _make_sc_agg(IN, True)
_sc_agg = _make_sc_agg(OUT, False)


@jax.jit
def kernel(x, edge_index, W1l, b1l, W1r, bn_gamma, bn_beta, bn_mean, bn_var,
           W2l, b2l, W2r):
    src = edge_index[0]
    dst = edge_index[1]
    src_p = jnp.concatenate([src, jnp.zeros((EP - E,), jnp.int32)]).reshape(
        EP // K, 1, K)
    dst_p = jnp.concatenate([dst, jnp.full((EP - E,), NP - 1, jnp.int32)]).reshape(
        EP // K, 1, K)
    epk = jnp.concatenate([src_p, dst_p], axis=1)
    x_p = jnp.pad(x, ((0, NP - N), (0, 0)))

    rstd = 1.0 / jnp.sqrt(bn_var + 1e-5)
    scale = (bn_gamma * rstd).reshape(1, H)
    shift = (bn_beta - bn_mean * bn_gamma * rstd).reshape(1, H)

    part1, degp = _sc_agg_deg(x_p, epk)
    deg3 = degp.reshape(NC, NB, BR).transpose(1, 0, 2)

    p, r, inv3 = _tc_stage1(
        part1, deg3, x_p, W1l.T, W1r.T, b1l.reshape(1, H), scale, shift,
        W2l.T, W2r.T, b2l.reshape(1, OUT),
    )

    (part2,) = _sc_agg(p, epk)
    out_p = _tc_stage2(part2, inv3, r)
    return out_p[:N]


# speedup vs baseline: 11.5939x; 2.9176x over previous
import functools

import jax
import jax.numpy as jnp
from jax import lax
from jax.experimental import pallas as pl
from jax.experimental.pallas import tpu as pltpu
from jax.experimental.pallas import tpu_sc as plsc

N = 10000
E = 320000
IN = 128
H = 256
OUT = 128

NC = 2
NS = 16
NW = NC * NS

NP = 10240
RPT = NP // NS
EP = 327680
EW = EP // NW
K = 128
CPW = EW // K
NBUF = 2
ZR = 128

BR = 512
NB = NP // BR


def _make_sc_agg(F, with_deg):
    mesh = plsc.VectorSubcoreMesh(
        core_axis_name="c", subcore_axis_name="s", num_cores=NC, num_subcores=NS
    )
    out_type = [jax.ShapeDtypeStruct((NC, NP, F), jnp.float32)]
    scratch = [
        pltpu.VMEM((3, 2, K), jnp.int32),
        pltpu.VMEM((2, K, F), jnp.float32),
        pltpu.VMEM_SHARED((NP, F), jnp.float32),
        pltpu.SemaphoreType.DMA,
        pltpu.SemaphoreType.DMA,
    ]
    if with_deg:
        out_type.append(jax.ShapeDtypeStruct((NC, NP), jnp.float32))
        scratch += [
            pltpu.SemaphoreType.DMA,
            pltpu.VMEM((K,), jnp.float32),
            pltpu.VMEM((RPT,), jnp.float32),
            pltpu.VMEM_SHARED((NP,), jnp.float32),
        ]

    def body(table, epk, *refs):
        if with_deg:
            out, dout, cidx, rows, acc, gsem, isem, dsem, ones, zdeg, dacc = refs
        else:
            out, cidx, rows, acc, gsem, isem = refs
        c = lax.axis_index("c")
        s = lax.axis_index("s")
        wid = s * NC + c

        def zfill(k, _):
            rows[0, k // (F // 16), pl.ds((k % (F // 16)) * 16, 16)] = jnp.zeros(
                (16,), jnp.float32
            )
            return 0

        lax.fori_loop(0, K * (F // 16), zfill, 0)
        for k in range(RPT // K):
            pltpu.sync_copy(rows.at[0], acc.at[pl.ds(s * RPT + k * K, K)])
        if with_deg:
            def ofill(k, _):
                ones[pl.ds(k * 16, 16)] = jnp.ones((16,), jnp.float32)
                return 0

            lax.fori_loop(0, K // 16, ofill, 0)

            def dzfill(k, _):
                zdeg[pl.ds(k * 16, 16)] = jnp.zeros((16,), jnp.float32)
                return 0

            lax.fori_loop(0, RPT // 16, dzfill, 0)
            pltpu.sync_copy(zdeg, dacc.at[pl.ds(s * RPT, RPT)])
        plsc.subcore_barrier()

        cb = wid * CPW

        pltpu.sync_copy(epk.at[cb], cidx.at[0])
        pltpu.async_copy(epk.at[cb + 1], cidx.at[1], isem)
        pltpu.async_copy(table.at[cidx.at[0, 0]], rows.at[0], gsem)

        def chunk(t, _):
            b = lax.rem(t, 2)
            bn = lax.rem(t + 1, 2)
            s0 = lax.rem(t, 3)
            s1 = lax.rem(t + 1, 3)
            s2 = lax.rem(t + 2, 3)
            tn1 = jnp.minimum(t + 1, CPW - 1)
            tn2 = jnp.minimum(t + 2, CPW - 1)
            pltpu.make_async_copy(table.at[cidx.at[s0, 0]], rows.at[b], gsem).wait()
            pltpu.make_async_copy(epk.at[cb + tn1], cidx.at[s1], isem).wait()
            pltpu.async_copy(table.at[cidx.at[s1, 0]], rows.at[bn], gsem)
            pltpu.async_copy(epk.at[cb + tn2], cidx.at[s2], isem)
            if with_deg:
                dsc = pltpu.async_copy(ones, dacc.at[cidx.at[s0, 1]], dsem, add=True)
            pltpu.sync_copy(rows.at[b], acc.at[cidx.at[s0, 1]], add=True)
            if with_deg:
                dsc.wait()
            return 0

        lax.fori_loop(0, CPW, chunk, 0)
        pltpu.make_async_copy(
            table.at[cidx.at[CPW % 3, 0]], rows.at[CPW % 2], gsem
        ).wait()
        pltpu.make_async_copy(
            epk.at[cb + CPW - 1], cidx.at[(CPW + 1) % 3], isem
        ).wait()
        plsc.subcore_barrier()

        pltpu.sync_copy(acc.at[pl.ds(s * RPT, RPT)], out.at[c, pl.ds(s * RPT, RPT)])
        if with_deg:
            pltpu.sync_copy(
                dacc.at[pl.ds(s * RPT, RPT)], dout.at[c, pl.ds(s * RPT, RPT)]
            )

    return pl.kernel(
        body, out_type=tuple(out_type), mesh=mesh, scratch_types=tuple(scratch)
    )


def _dot(a, b):
    return jnp.dot(a, b, preferred_element_type=jnp.float32,
                   precision=lax.Precision.HIGHEST)


def _tc_stage1(part1, deg3, x, w1lT, w1rT, b1l, scale, shift, w2lT, w2rT, b2l):
    def body(p_ref, d_ref, x_ref, w1l_ref, w1r_ref, b1_ref, sc_ref, sh_ref,
             w2l_ref, w2r_ref, b2_ref, pout_ref, rout_ref, inv_ref):
        deg = d_ref[0, 0] + d_ref[0, 1]
        inv = 1.0 / jnp.maximum(deg, 1.0)
        inv_ref[...] = inv.reshape(1, 1, BR)
        agg = (p_ref[0] + p_ref[1]) * inv[:, None]
        t = _dot(agg, w1l_ref[...]) + _dot(x_ref[...], w1r_ref[...]) + b1_ref[...]
        h = jnp.maximum(t * sc_ref[...] + sh_ref[...], 0.0)
        pout_ref[...] = _dot(h, w2l_ref[...])
        rout_ref[...] = _dot(h, w2r_ref[...]) + b2_ref[...]

    full = lambda i: (0, 0)
    return pl.pallas_call(
        body,
        grid=(NB,),
        in_specs=[
            pl.BlockSpec((NC, BR, IN), lambda i: (0, i, 0)),
            pl.BlockSpec((1, NC, BR), lambda i: (i, 0, 0)),
            pl.BlockSpec((BR, IN), lambda i: (i, 0)),
            pl.BlockSpec((IN, H), full),
            pl.BlockSpec((IN, H), full),
            pl.BlockSpec((1, H), full),
            pl.BlockSpec((1, H), full),
            pl.BlockSpec((1, H), full),
            pl.BlockSpec((H, OUT), full),
            pl.BlockSpec((H, OUT), full),
            pl.BlockSpec((1, OUT), full),
        ],
        out_specs=[
            pl.BlockSpec((BR, OUT), lambda i: (i, 0)),
            pl.BlockSpec((BR, OUT), lambda i: (i, 0)),
            pl.BlockSpec((1, 1, BR), lambda i: (i, 0, 0)),
        ],
        out_shape=[
            jax.ShapeDtypeStruct((NP, OUT), jnp.float32),
            jax.ShapeDtypeStruct((NP, OUT), jnp.float32),
            jax.ShapeDtypeStruct((NB, 1, BR), jnp.float32),
        ],
    )(part1, deg3, x, w1lT, w1rT, b1l, scale, shift, w2lT, w2rT, b2l)


def _tc_stage2(part2, inv3, r):
    def body(p_ref, i_ref, r_ref, o_ref):
        inv = i_ref[0, 0]
        o_ref[...] = (p_ref[0] + p_ref[1]) * inv[:, None] + r_ref[...]

    return pl.pallas_call(
        body,
        grid=(NB,),
        in_specs=[
            pl.BlockSpec((NC, BR, OUT), lambda i: (0, i, 0)),
            pl.BlockSpec((1, 1, BR), lambda i: (i, 0, 0)),
            pl.BlockSpec((BR, OUT), lambda i: (i, 0)),
        ],
        out_specs=pl.BlockSpec((BR, OUT), lambda i: (i, 0)),
        out_shape=jax.ShapeDtypeStruct((NP, OUT), jnp.float32),
    )(part2, inv3, r)


_sc_agg_deg = _make_sc_agg(IN, True)
_sc_agg = _make_sc_agg(OUT, False)


@jax.jit
def kernel(x, edge_index, W1l, b1l, W1r, bn_gamma, bn_beta, bn_mean, bn_var,
           W2l, b2l, W2r):
    src = edge_index[0]
    dst = edge_index[1]
    pad_i = jnp.arange(EP - E, dtype=jnp.int32)
    src_p = jnp.concatenate([src, pad_i % N]).reshape(EP // K, 1, K)
    dst_p = jnp.concatenate([dst, N + pad_i % (NP - N)]).reshape(EP // K, 1, K)
    epk = jnp.concatenate([src_p, dst_p], axis=1)
    x_p = jnp.pad(x, ((0, NP - N), (0, 0)))

    rstd = 1.0 / jnp.sqrt(bn_var + 1e-5)
    scale = (bn_gamma * rstd).reshape(1, H)
    shift = (bn_beta - bn_mean * bn_gamma * rstd).reshape(1, H)

    part1, degp = _sc_agg_deg(x_p, epk)
    deg3 = degp.reshape(NC, NB, BR).transpose(1, 0, 2)

    p, r, inv3 = _tc_stage1(
        part1, deg3, x_p, W1l.T, W1r.T, b1l.reshape(1, H), scale, shift,
        W2l.T, W2r.T, b2l.reshape(1, OUT),
    )

    (part2,) = _sc_agg(p, epk)
    out_p = _tc_stage2(part2, inv3, r)
    return out_p[:N]


# speedup vs baseline: 13.8229x; 1.1923x over previous
import jax
import jax.numpy as jnp
from jax import lax
from jax.experimental import pallas as pl
from jax.experimental.pallas import tpu as pltpu
from jax.experimental.pallas import tpu_sc as plsc

N = 10000
E = 320000
IN = 128
H = 256
OUT = 128

NC = 2
NS = 16
NW = NC * NS

NP = 10240
RPT = NP // NS
EP = 327680
EW = EP // NW
K = 128
CPW = EW // K

BR = 1024
BR1 = 1024
BR2 = 2048
NB = NP // BR


def _make_sc_agg(F, with_deg, dtype=jnp.float32):
    mesh = plsc.VectorSubcoreMesh(
        core_axis_name="c", subcore_axis_name="s", num_cores=NC, num_subcores=NS
    )
    out_type = [jax.ShapeDtypeStruct((NC, NP, F), dtype)]
    scratch = [
        pltpu.VMEM((3, 2, K), jnp.int32),
        pltpu.VMEM((2, K, F), dtype),
        pltpu.VMEM_SHARED((NP, F), dtype),
        pltpu.SemaphoreType.DMA,
        pltpu.SemaphoreType.DMA,
    ]
    if with_deg:
        out_type.append(jax.ShapeDtypeStruct((NC, NP), jnp.float32))
        scratch += [
            pltpu.SemaphoreType.DMA,
            pltpu.VMEM((K,), jnp.float32),
            pltpu.VMEM((RPT,), jnp.float32),
            pltpu.VMEM_SHARED((NP,), jnp.float32),
        ]

    def body(table, epk, *refs):
        if with_deg:
            out, dout, cidx, rows, acc, gsem, isem, dsem, ones, zdeg, dacc = refs
        else:
            out, cidx, rows, acc, gsem, isem = refs
        c = lax.axis_index("c")
        s = lax.axis_index("s")
        wid = s * NC + c

        pltpu.sync_copy(table.at[pl.ds(N, K)], rows.at[0])
        for k in range(RPT // K):
            pltpu.sync_copy(rows.at[0], acc.at[pl.ds(s * RPT + k * K, K)])
        if with_deg:
            def ofill(k, _):
                ones[pl.ds(k * 16, 16)] = jnp.ones((16,), jnp.float32)
                return 0

            lax.fori_loop(0, K // 16, ofill, 0)

            def dzfill(k, _):
                zdeg[pl.ds(k * 16, 16)] = jnp.zeros((16,), jnp.float32)
                return 0

            lax.fori_loop(0, RPT // 16, dzfill, 0)
            pltpu.sync_copy(zdeg, dacc.at[pl.ds(s * RPT, RPT)])
        cb = wid * CPW

        pltpu.sync_copy(epk.at[cb], cidx.at[0])
        pltpu.async_copy(epk.at[cb + 1], cidx.at[1], isem)
        pltpu.async_copy(table.at[cidx.at[0, 0]], rows.at[0], gsem)
        plsc.subcore_barrier()

        def chunk(t, _):
            b = lax.rem(t, 2)
            bn = lax.rem(t + 1, 2)
            s0 = lax.rem(t, 3)
            s1 = lax.rem(t + 1, 3)
            s2 = lax.rem(t + 2, 3)
            tn1 = jnp.minimum(t + 1, CPW - 1)
            tn2 = jnp.minimum(t + 2, CPW - 1)
            pltpu.make_async_copy(table.at[cidx.at[s0, 0]], rows.at[b], gsem).wait()
            pltpu.make_async_copy(epk.at[cb + tn1], cidx.at[s1], isem).wait()
            pltpu.async_copy(table.at[cidx.at[s1, 0]], rows.at[bn], gsem)
            pltpu.async_copy(epk.at[cb + tn2], cidx.at[s2], isem)
            if with_deg:
                dsc = pltpu.async_copy(ones, dacc.at[cidx.at[s0, 1]], dsem, add=True)
            pltpu.sync_copy(rows.at[b], acc.at[cidx.at[s0, 1]], add=True)
            if with_deg:
                dsc.wait()
            return 0

        lax.fori_loop(0, CPW, chunk, 0)
        pltpu.make_async_copy(
            table.at[cidx.at[CPW % 3, 0]], rows.at[CPW % 2], gsem
        ).wait()
        pltpu.make_async_copy(
            epk.at[cb + CPW - 1], cidx.at[(CPW + 1) % 3], isem
        ).wait()
        plsc.subcore_barrier()

        pltpu.sync_copy(acc.at[pl.ds(s * RPT, RPT)], out.at[c, pl.ds(s * RPT, RPT)])
        if with_deg:
            pltpu.sync_copy(
                dacc.at[pl.ds(s * RPT, RPT)], dout.at[c, pl.ds(s * RPT, RPT)]
            )

    return pl.kernel(
        body, out_type=tuple(out_type), mesh=mesh, scratch_types=tuple(scratch)
    )


def _dot(a, b):
    return jnp.dot(a, b, preferred_element_type=jnp.float32,
                   precision=lax.Precision.DEFAULT)


def _tc_stage1(part1, deg3, x, w1lT, w1rT, b1l, scale, shift, w2lT, w2rT, b2l):
    def body(p_ref, d_ref, x_ref, w1l_ref, w1r_ref, b1_ref, sc_ref, sh_ref,
             w2l_ref, w2r_ref, b2_ref, pout_ref, rout_ref, inv_ref):
        i = pl.program_id(0)
        deg = d_ref[0, 0] + d_ref[0, 1]
        inv = 1.0 / jnp.maximum(deg, 1.0)
        inv_ref[...] = inv.reshape(1, 1, BR)
        agg = (p_ref[0].astype(jnp.float32)
               + p_ref[1].astype(jnp.float32)) * inv[:, None]
        t = _dot(agg, w1l_ref[...]) + _dot(x_ref[...], w1r_ref[...]) + b1_ref[...]
        h = jnp.maximum(t * sc_ref[...] + sh_ref[...], 0.0)
        rix = i * BR1 + lax.broadcasted_iota(jnp.int32, (BR1, OUT), 0)
        pv = _dot(h, w2l_ref[...])
        pout_ref[...] = jnp.where(rix < N, pv, 0.0)
        rout_ref[...] = _dot(h, w2r_ref[...]) + b2_ref[...]

    full = lambda i: (0, 0)
    return pl.pallas_call(
        body,
        grid=(NP // BR1,),
        in_specs=[
            pl.BlockSpec((NC, BR1, IN), lambda i: (0, i, 0)),
            pl.BlockSpec((1, NC, BR), lambda i: (i, 0, 0)),
            pl.BlockSpec((BR1, IN), lambda i: (i, 0)),
            pl.BlockSpec((IN, H), full),
            pl.BlockSpec((IN, H), full),
            pl.BlockSpec((1, H), full),
            pl.BlockSpec((1, H), full),
            pl.BlockSpec((1, H), full),
            pl.BlockSpec((H, OUT), full),
            pl.BlockSpec((H, OUT), full),
            pl.BlockSpec((1, OUT), full),
        ],
        out_specs=[
            pl.BlockSpec((BR1, OUT), lambda i: (i, 0)),
            pl.BlockSpec((BR1, OUT), lambda i: (i, 0)),
            pl.BlockSpec((1, 1, BR), lambda i: (i, 0, 0)),
        ],
        out_shape=[
            jax.ShapeDtypeStruct((NP, OUT), jnp.float32),
            jax.ShapeDtypeStruct((NP, OUT), jnp.float32),
            jax.ShapeDtypeStruct((NB, 1, BR), jnp.float32),
        ],
    )(part1, deg3, x, w1lT, w1rT, b1l, scale, shift, w2lT, w2rT, b2l)


def _tc_stage2(part2, inv3, r):
    def body(p_ref, i_ref, r_ref, o_ref):
        g = BR2 // BR
        inv = i_ref[:, 0]
        ps = (p_ref[0].astype(jnp.float32)
              + p_ref[1].astype(jnp.float32)).reshape(g, BR, OUT)
        rs = r_ref[...].reshape(g, BR, OUT)
        o_ref[...] = (ps * inv[:, :, None] + rs).reshape(BR2, OUT)

    return pl.pallas_call(
        body,
        grid=(NP // BR2,),
        in_specs=[
            pl.BlockSpec((NC, BR2, OUT), lambda i: (0, i, 0)),
            pl.BlockSpec((BR2 // BR, 1, BR), lambda i: (i, 0, 0)),
            pl.BlockSpec((BR2, OUT), lambda i: (i, 0)),
        ],
        out_specs=pl.BlockSpec((BR2, OUT), lambda i: (i, 0)),
        out_shape=jax.ShapeDtypeStruct((N, OUT), jnp.float32),
    )(part2, inv3, r)


_sc_agg_deg = _make_sc_agg(IN, True)
_sc_agg = _make_sc_agg(OUT, False)


@jax.jit
def kernel(x, edge_index, W1l, b1l, W1r, bn_gamma, bn_beta, bn_mean, bn_var,
           W2l, b2l, W2r):
    src = edge_index[0]
    dst = edge_index[1]
    pad_i = jnp.arange(EP - E, dtype=jnp.int32)
    src_p = jnp.concatenate([src, pad_i % N]).reshape(EP // K, 1, K)
    dst_p = jnp.concatenate([dst, N + pad_i % (NP - N)]).reshape(EP // K, 1, K)
    epk = jnp.concatenate([src_p, dst_p], axis=1)
    x_p = jnp.pad(x, ((0, NP - N), (0, 0)))

    rstd = 1.0 / jnp.sqrt(bn_var + 1e-5)
    scale = (bn_gamma * rstd).reshape(1, H)
    shift = (bn_beta - bn_mean * bn_gamma * rstd).reshape(1, H)

    part1, degp = _sc_agg_deg(x_p, epk)
    deg3 = degp.reshape(NC, NB, BR).transpose(1, 0, 2)

    p, r, inv3 = _tc_stage1(
        part1, deg3, x_p, W1l.T, W1r.T, b1l.reshape(1, H), scale, shift,
        W2l.T, W2r.T, b2l.reshape(1, OUT),
    )

    (part2,) = _sc_agg(p, epk)
    return _tc_stage2(part2, inv3, r)
